# baseline (speedup 1.0000x reference)
import functools

import jax
import jax.numpy as jnp
from jax import lax
from jax.experimental import pallas as pl
from jax.experimental.pallas import tpu as pltpu
from jax.experimental.pallas import tpu_sc as plsc

_NUM_EMB = 1024
_EMB_SIZE = 8
_NB = 16384
_NSLOT = 26

_info = plsc.get_sparse_core_info()
_NC, _NS, _L = _info.num_cores, _info.num_subcores, _info.num_lanes
_NW = _NC * _NS
_BW = _NB // _NW
_GROUPS = _BW // _L


@functools.partial(
    pl.kernel,
    out_type=jax.ShapeDtypeStruct((_NSLOT, _EMB_SIZE, _NB), jnp.float32),
    mesh=plsc.VectorSubcoreMesh(core_axis_name="c", subcore_axis_name="s"),
    compiler_params=pltpu.CompilerParams(needs_layout_passes=False),
    scratch_types=[
        pltpu.VMEM((_EMB_SIZE, _NUM_EMB), jnp.float32),
        pltpu.VMEM((_NSLOT, _BW), jnp.int32),
        pltpu.VMEM((_NSLOT, _EMB_SIZE, _BW), jnp.float32),
        pltpu.SemaphoreType.DMA,
        pltpu.SemaphoreType.DMA,
    ],
)
def _gather_kernel(keys_hbm, table_hbm, out_hbm, table_v, keys_v, out_v,
                   in_sem, out_sem):
    wid = lax.axis_index("s") * _NC + lax.axis_index("c")
    b0 = wid * _BW

    tbl_cp = pltpu.async_copy(table_hbm, table_v, in_sem)
    key_cp = pltpu.async_copy(keys_hbm.at[:, pl.ds(b0, _BW)], keys_v, in_sem)
    tbl_cp.wait()
    key_cp.wait()

    iota = lax.iota(jnp.int32, _L)
    cvecs = [jnp.full((_L,), c, jnp.int32) for c in range(_EMB_SIZE)]

    def _slot(s, _):
        sfull = jnp.full((_L,), s, jnp.int32)

        @plsc.parallel_loop(0, _GROUPS, unroll=2)
        def _body(g):
            bvec = g * _L + iota
            keys16 = plsc.load_gather(keys_v, [sfull, bvec])
            for c in range(_EMB_SIZE):
                col = plsc.load_gather(table_v, [cvecs[c], keys16])
                plsc.store_scatter(out_v, [sfull, cvecs[c], bvec], col)

        pltpu.async_copy(out_v.at[s], out_hbm.at[s, :, pl.ds(b0, _BW)],
                         out_sem)
        return 0

    lax.fori_loop(0, _NSLOT, _slot, 0)
    pltpu.make_async_copy(out_hbm.at[:, :, pl.ds(b0, _BW)], out_v,
                          out_sem).wait()


def kernel(keys, table):
    keys_t = keys.T
    table_t = table.T
    p = _gather_kernel(keys_t, table_t)
    return p.transpose(2, 0, 1)


# speedup vs baseline: 1.0022x; 1.0022x over previous
import functools

import jax
import jax.numpy as jnp
from jax import lax
from jax.experimental import pallas as pl
from jax.experimental.pallas import tpu as pltpu
from jax.experimental.pallas import tpu_sc as plsc

_NUM_EMB = 1024
_EMB_SIZE = 8
_NB = 16384
_NSLOT = 26

_info = plsc.get_sparse_core_info()
_NC, _NS, _L = _info.num_cores, _info.num_subcores, _info.num_lanes
_NW = _NC * _NS
_BW = _NB // _NW
_GROUPS = _BW // _L


@functools.partial(
    pl.kernel,
    out_type=jax.ShapeDtypeStruct((_NSLOT, _EMB_SIZE, _NB), jnp.float32),
    mesh=plsc.VectorSubcoreMesh(core_axis_name="c", subcore_axis_name="s"),
    compiler_params=pltpu.CompilerParams(needs_layout_passes=False),
    scratch_types=[
        pltpu.VMEM((_EMB_SIZE, _NUM_EMB), jnp.float32),
        pltpu.VMEM((_NSLOT, _BW), jnp.int32),
        pltpu.VMEM((_NSLOT, _EMB_SIZE, _BW), jnp.float32),
        pltpu.SemaphoreType.DMA,
        pltpu.SemaphoreType.DMA,
    ],
)
def _gather_kernel(keys_hbm, table_hbm, out_hbm, table_v, keys_v, out_v,
                   in_sem, out_sem):
    wid = lax.axis_index("s") * _NC + lax.axis_index("c")
    b0 = wid * _BW

    tbl_cp = pltpu.async_copy(table_hbm, table_v, in_sem)
    key_cp = pltpu.async_copy(keys_hbm.at[:, pl.ds(b0, _BW)], keys_v, in_sem)
    tbl_cp.wait()
    key_cp.wait()

    iota = lax.iota(jnp.int32, _L)
    cvecs = [jnp.full((_L,), c, jnp.int32) for c in range(_EMB_SIZE)]

    def _slot(s, _):
        sfull = jnp.full((_L,), s, jnp.int32)

        @plsc.parallel_loop(0, _GROUPS, unroll=4)
        def _body(g):
            bvec = g * _L + iota
            keys16 = plsc.load_gather(keys_v, [sfull, bvec])
            for c in range(_EMB_SIZE):
                col = plsc.load_gather(table_v, [cvecs[c], keys16])
                plsc.store_scatter(out_v, [sfull, cvecs[c], bvec], col)

        pltpu.async_copy(out_v.at[s], out_hbm.at[s, :, pl.ds(b0, _BW)],
                         out_sem)
        return 0

    lax.fori_loop(0, _NSLOT, _slot, 0)
    pltpu.make_async_copy(out_hbm.at[:, :, pl.ds(b0, _BW)], out_v,
                          out_sem).wait()


def kernel(keys, table):
    keys_t = keys.T
    table_t = table.T
    p = _gather_kernel(keys_t, table_t)
    return p.transpose(2, 0, 1)


# speedup vs baseline: 1.0035x; 1.0013x over previous
import functools

import jax
import jax.numpy as jnp
from jax import lax
from jax.experimental import pallas as pl
from jax.experimental.pallas import tpu as pltpu
from jax.experimental.pallas import tpu_sc as plsc

_NUM_EMB = 1024
_EMB_SIZE = 8
_NB = 16384
_NSLOT = 26

_info = plsc.get_sparse_core_info()
_NC, _NS, _L = _info.num_cores, _info.num_subcores, _info.num_lanes
_NW = _NC * _NS
_BW = _NB // _NW
_GROUPS = _BW // _L


@functools.partial(
    pl.kernel,
    out_type=jax.ShapeDtypeStruct((_NSLOT, _EMB_SIZE, _NB), jnp.float32),
    mesh=plsc.VectorSubcoreMesh(core_axis_name="c", subcore_axis_name="s"),
    compiler_params=pltpu.CompilerParams(needs_layout_passes=False,
                                         skip_device_barrier=True),
    scratch_types=[
        pltpu.VMEM((_EMB_SIZE, _NUM_EMB), jnp.float32),
        pltpu.VMEM((_NSLOT, _BW), jnp.int32),
        pltpu.VMEM((_NSLOT, _EMB_SIZE, _BW), jnp.float32),
        pltpu.SemaphoreType.DMA,
        pltpu.SemaphoreType.DMA,
    ],
)
def _gather_kernel(keys_hbm, table_hbm, out_hbm, table_v, keys_v, out_v,
                   in_sem, out_sem):
    wid = lax.axis_index("s") * _NC + lax.axis_index("c")
    b0 = wid * _BW

    tbl_cp = pltpu.async_copy(table_hbm, table_v, in_sem)
    key_cp = pltpu.async_copy(keys_hbm.at[:, pl.ds(b0, _BW)], keys_v, in_sem)
    tbl_cp.wait()
    key_cp.wait()

    iota = lax.iota(jnp.int32, _L)
    cvecs = [jnp.full((_L,), c, jnp.int32) for c in range(_EMB_SIZE)]

    def _slot(s, _):
        sfull = jnp.full((_L,), s, jnp.int32)

        @plsc.parallel_loop(0, _GROUPS, unroll=4)
        def _body(g):
            bvec = g * _L + iota
            keys16 = plsc.load_gather(keys_v, [sfull, bvec])
            for c in range(_EMB_SIZE):
                col = plsc.load_gather(table_v, [cvecs[c], keys16])
                plsc.store_scatter(out_v, [sfull, cvecs[c], bvec], col)

        pltpu.async_copy(out_v.at[s], out_hbm.at[s, :, pl.ds(b0, _BW)],
                         out_sem)
        return 0

    lax.fori_loop(0, _NSLOT, _slot, 0)
    pltpu.make_async_copy(out_hbm.at[:, :, pl.ds(b0, _BW)], out_v,
                          out_sem).wait()


def kernel(keys, table):
    keys_t = keys.T
    table_t = table.T
    p = _gather_kernel(keys_t, table_t)
    return p.transpose(2, 0, 1)


# speedup vs baseline: 1.0047x; 1.0012x over previous
import functools

import jax
import jax.numpy as jnp
from jax import lax
from jax.experimental import pallas as pl
from jax.experimental.pallas import tpu as pltpu
from jax.experimental.pallas import tpu_sc as plsc

_NUM_EMB = 1024
_EMB_SIZE = 8
_NB = 16384
_NSLOT = 26

_info = plsc.get_sparse_core_info()
_NC, _NS, _L = _info.num_cores, _info.num_subcores, _info.num_lanes
_NW = _NC * _NS
_BW = _NB // _NW
_GROUPS = _BW // _L


@functools.partial(
    pl.kernel,
    out_type=jax.ShapeDtypeStruct((_NSLOT, _EMB_SIZE, _NB), jnp.float32),
    mesh=plsc.VectorSubcoreMesh(core_axis_name="c", subcore_axis_name="s"),
    compiler_params=pltpu.CompilerParams(needs_layout_passes=False),
    scratch_types=[
        pltpu.VMEM((_EMB_SIZE, _NUM_EMB), jnp.float32),
        pltpu.VMEM((_NSLOT, _BW), jnp.int32),
        pltpu.VMEM((_NSLOT, _EMB_SIZE, _BW), jnp.float32),
        pltpu.SemaphoreType.DMA,
        pltpu.SemaphoreType.DMA,
    ],
)
def _gather_kernel(keys_hbm, table_hbm, out_hbm, table_v, keys_v, out_v,
                   in_sem, out_sem):
    wid = lax.axis_index("s") * _NC + lax.axis_index("c")
    b0 = wid * _BW

    tbl_cp = pltpu.async_copy(table_hbm, table_v, in_sem)
    key_cp = pltpu.async_copy(keys_hbm.at[:, pl.ds(b0, _BW)], keys_v, in_sem)
    tbl_cp.wait()
    key_cp.wait()

    iota = lax.iota(jnp.int32, _L)
    cvecs = [jnp.full((_L,), c, jnp.int32) for c in range(_EMB_SIZE)]

    def _slot(s, _):
        sfull = jnp.full((_L,), s, jnp.int32)

        @plsc.parallel_loop(0, _GROUPS, unroll=4)
        def _body(g):
            bvec = g * _L + iota
            keys16 = plsc.load_gather(keys_v, [sfull, bvec])
            for c in range(_EMB_SIZE):
                col = plsc.load_gather(table_v, [cvecs[c], keys16])
                plsc.store_scatter(out_v, [sfull, cvecs[c], bvec], col)

        pltpu.async_copy(out_v.at[s], out_hbm.at[s, :, pl.ds(b0, _BW)],
                         out_sem)
        return 0

    lax.fori_loop(0, _NSLOT, _slot, 0)
    pltpu.make_async_copy(out_hbm.at[:, :, pl.ds(b0, _BW)], out_v,
                          out_sem).wait()


def kernel(keys, table):
    keys_t = keys.T
    table_t = table.T
    p = _gather_kernel(keys_t, table_t)
    return p.transpose(2, 0, 1)
